# Initial kernel scaffold; baseline (speedup 1.0000x reference)
import jax
import jax.numpy as jnp
from jax.experimental import pallas as pl


def kernel(inputs, selected_edges):
    raise NotImplementedError("write your pallas kernel here")


# speedup vs baseline: 30.1024x; 30.1024x over previous
import functools

import jax
import jax.numpy as jnp
from jax import lax
from jax.experimental import pallas as pl
from jax.experimental.pallas import tpu as pltpu
from jax.experimental.pallas import tpu_sc as plsc

_B, _N, _D, _E = 4, 10000, 128, 320000
_NC, _NS = 2, 16
_NW = _NC * _NS
_EPW = _E // _NW
_C = 80
_GPW = _EPW // _C


def _gather_body(table, idx_i, idx_j, out_i, out_j,
                 idx_i_v, idx_j_v, rows_i, rows_j, sem_i, sem_j):
    wid = lax.axis_index("s") * _NC + lax.axis_index("c")
    pltpu.sync_copy(idx_i.at[wid], idx_i_v)
    pltpu.sync_copy(idx_j.at[wid], idx_j_v)

    def chunk(g, carry):
        e_off = wid * _EPW + g * _C
        cp_i = pltpu.async_copy(table.at[idx_i_v.at[g]], rows_i, sem_i)
        cp_j = pltpu.async_copy(table.at[idx_j_v.at[g]], rows_j, sem_j)
        cp_i.wait()
        pltpu.sync_copy(rows_i, out_i.at[pl.ds(e_off, _C)])
        cp_j.wait()
        pltpu.sync_copy(rows_j, out_j.at[pl.ds(e_off, _C)])
        return carry

    lax.fori_loop(0, _GPW, chunk, 0)


@jax.jit
def _gather(table, idx_i, idx_j):
    mesh = plsc.VectorSubcoreMesh(
        core_axis_name="c", subcore_axis_name="s",
        num_cores=_NC, num_subcores=_NS,
    )
    return pl.kernel(
        _gather_body,
        out_type=(
            jax.ShapeDtypeStruct((_E, _D), jnp.float32),
            jax.ShapeDtypeStruct((_E, _D), jnp.float32),
        ),
        mesh=mesh,
        scratch_types=[
            pltpu.VMEM((_GPW, _C), jnp.int32),
            pltpu.VMEM((_GPW, _C), jnp.int32),
            pltpu.VMEM((_C, _D), jnp.float32),
            pltpu.VMEM((_C, _D), jnp.float32),
            pltpu.SemaphoreType.DMA,
            pltpu.SemaphoreType.DMA,
        ],
    )(table, idx_i, idx_j)


def kernel(inputs, selected_edges):
    table = inputs.reshape(_B * _N, _D)
    idx_i = selected_edges[:, 4].reshape(_NW, _GPW, _C)
    idx_j = selected_edges[:, 5].reshape(_NW, _GPW, _C)
    return _gather(table, idx_i, idx_j)


# speedup vs baseline: 38.7050x; 1.2858x over previous
import jax
import jax.numpy as jnp
from jax import lax
from jax.experimental import pallas as pl
from jax.experimental.pallas import tpu as pltpu
from jax.experimental.pallas import tpu_sc as plsc

_B, _N, _D, _E = 4, 10000, 128, 320000
_NC, _NS = 2, 16
_NW = _NC * _NS
_EPW = _E // _NW
_C = 80
_GPW = _EPW // _C
_R = 5
_NBLK = _GPW // _R


def _gather_body(table, idx_i, idx_j, out_i, out_j,
                 idx_i_v, idx_j_v, rows_i, rows_j, *sems):
    gs = (sems[0:_R], sems[_R:2 * _R])
    ss = (sems[2 * _R:3 * _R], sems[3 * _R:4 * _R])
    idx_v = (idx_i_v, idx_j_v)
    rows = (rows_i, rows_j)
    outs = (out_i, out_j)

    wid = lax.axis_index("s") * _NC + lax.axis_index("c")
    ebase = wid * _EPW
    pltpu.sync_copy(idx_i.at[wid], idx_i_v)
    pltpu.sync_copy(idx_j.at[wid], idx_j_v)

    def start_gather(ep, b, g):
        pltpu.async_copy(
            table.at[idx_v[ep].at[pl.ds(g * _C, _C)]], rows[ep].at[b],
            gs[ep][b])

    def wait_gather(ep, b, g):
        pltpu.make_async_copy(
            table.at[idx_v[ep].at[pl.ds(g * _C, _C)]], rows[ep].at[b],
            gs[ep][b]).wait()

    for b in range(_R):
        for ep in range(2):
            start_gather(ep, b, b)

    def block(t, carry):
        cps = []
        for b in range(_R):
            g = t * _R + b
            for ep in range(2):
                wait_gather(ep, b, g)
                cps.append(pltpu.async_copy(
                    rows[ep].at[b],
                    outs[ep].at[pl.ds(ebase + g * _C, _C)],
                    ss[ep][b]))
        for b in range(_R):
            for ep in range(2):
                cps[2 * b + ep].wait()

            @pl.when(t < _NBLK - 1)
            def _():
                g2 = (t + 1) * _R + b
                for ep in range(2):
                    start_gather(ep, b, g2)
        return carry

    lax.fori_loop(0, _NBLK, block, 0)


@jax.jit
def _gather(table, idx_i, idx_j):
    mesh = plsc.VectorSubcoreMesh(
        core_axis_name="c", subcore_axis_name="s",
        num_cores=_NC, num_subcores=_NS,
    )
    return pl.kernel(
        _gather_body,
        out_type=(
            jax.ShapeDtypeStruct((_E, _D), jnp.float32),
            jax.ShapeDtypeStruct((_E, _D), jnp.float32),
        ),
        mesh=mesh,
        scratch_types=[
            pltpu.VMEM((_EPW,), jnp.int32),
            pltpu.VMEM((_EPW,), jnp.int32),
            pltpu.VMEM((_R, _C, _D), jnp.float32),
            pltpu.VMEM((_R, _C, _D), jnp.float32),
        ] + [pltpu.SemaphoreType.DMA] * (4 * _R),
    )(table, idx_i, idx_j)


def kernel(inputs, selected_edges):
    table = inputs.reshape(_B * _N, _D)
    idx_i = selected_edges[:, 4].reshape(_NW, _EPW)
    idx_j = selected_edges[:, 5].reshape(_NW, _EPW)
    return _gather(table, idx_i, idx_j)
